# speedup vs baseline: 1.0262x; 1.0262x over previous
import functools

import jax
import jax.numpy as jnp
import numpy as np
from jax.experimental import pallas as pl
from jax.experimental.pallas import tpu as pltpu

N_NODES = 10000
N_EDGES = 320000
ROW_BLK = 1000


def _proj_kernel(x_ref, w_ref, a_ref, h_ref, ap_ref):
    h = jnp.dot(x_ref[...], w_ref[...], preferred_element_type=jnp.float32)
    h_ref[...] = h
    ap_ref[...] = jnp.dot(h, a_ref[...], preferred_element_type=jnp.float32)


def _proj(x, W, A):
    n, d_in = x.shape
    d_out = W.shape[1]
    grid = (n // ROW_BLK,)
    h, ap = pl.pallas_call(
        _proj_kernel,
        grid=grid,
        in_specs=[
            pl.BlockSpec((ROW_BLK, d_in), lambda i: (i, 0)),
            pl.BlockSpec((d_in, d_out), lambda i: (0, 0)),
            pl.BlockSpec((d_out, 128), lambda i: (0, 0)),
        ],
        out_specs=[
            pl.BlockSpec((ROW_BLK, d_out), lambda i: (i, 0)),
            pl.BlockSpec((ROW_BLK, 128), lambda i: (i, 0)),
        ],
        out_shape=[
            jax.ShapeDtypeStruct((n, d_out), jnp.float32),
            jax.ShapeDtypeStruct((n, 128), jnp.float32),
        ],
    )(x, W, A)
    return h, ap


def _att_matrix(att_src, att_dst, heads, ch):
    d_out = heads * ch
    A = jnp.zeros((d_out, 128), jnp.float32)
    hh = jnp.arange(heads)
    cc = jnp.arange(ch)
    rows = (hh[:, None] * ch + cc[None, :]).reshape(-1)
    cols_src = jnp.repeat(hh, ch)
    A = A.at[rows, cols_src].set(att_src.reshape(-1))
    A = A.at[rows, cols_src + 8].set(att_dst.reshape(-1))
    return A


def _gat_layer(x, ei, W, att_src, att_dst, bias, heads, ch):
    A = _att_matrix(att_src, att_dst, heads, ch)
    h, ap = _proj(x, W, A)
    a_src = ap[:, :heads]
    a_dst = ap[:, 8:8 + heads]
    src, dst = ei[0], ei[1]
    alpha = a_src[src] + a_dst[dst]
    alpha = jax.nn.leaky_relu(alpha, negative_slope=0.2)
    amax = jax.ops.segment_max(alpha, dst, num_segments=N_NODES)
    amax = jnp.where(jnp.isfinite(amax), amax, 0.0)
    ealpha = jnp.exp(alpha - amax[dst])
    denom = jax.ops.segment_sum(ealpha, dst, num_segments=N_NODES)
    alpha_n = ealpha / (denom[dst] + 1e-16)
    hh = h.reshape(N_NODES, heads, ch)
    msg = hh[src] * alpha_n[:, :, None]
    out = jax.ops.segment_sum(msg, dst, num_segments=N_NODES)
    return out.reshape(N_NODES, heads * ch) + bias


def kernel(edge_index, node_emb, W1, att_src1, att_dst1, b1, W2, att_src2,
           att_dst2, b2):
    loops = jnp.arange(N_NODES, dtype=edge_index.dtype)
    ei = jnp.concatenate([edge_index, jnp.stack([loops, loops])], axis=1)
    ei = ei.astype(jnp.int32)
    x = jax.nn.elu(_gat_layer(node_emb, ei, W1, att_src1, att_dst1, b1, 8, 16))
    return _gat_layer(x, ei, W2, att_src2, att_dst2, b2, 1, 64)


# speedup vs baseline: 41.6983x; 40.6344x over previous
import functools

import jax
import jax.numpy as jnp
from jax import lax
from jax.experimental import pallas as pl
from jax.experimental.pallas import tpu as pltpu
from jax.experimental.pallas import tpu_sc as plsc

N_NODES = 10000
N_EDGES = 320000
E_TOT = N_EDGES + N_NODES
NW = 32
K_EDGE = 128
T_EDGE = 10368
E_PAD = NW * T_EDGE
N_ACC = 10112
TRASH = 10000
ROW_BLK = 1000


def _prep_body(nblk, x_ref, w_ref, a_ref, hs_ref, adst_ref, cvec_ref, mx_ref):
    i = pl.program_id(0)
    d = w_ref.shape[1]
    h = jnp.dot(x_ref[...], w_ref[...], preferred_element_type=jnp.float32)
    ap = jnp.dot(h, a_ref[...], preferred_element_type=jnp.float32)
    hs_ref[:, :d] = h
    hs_ref[:, d:d + 16] = ap[:, 0:16]
    adst_ref[...] = ap[:, 16:32]
    blkmax = jnp.max(ap[:, 0:16], axis=0, keepdims=True)

    @pl.when(i == 0)
    def _():
        mx_ref[...] = blkmax

    @pl.when(i > 0)
    def _():
        mx_ref[...] = jnp.maximum(mx_ref[...], blkmax)

    cvec_ref[...] = jnp.concatenate(
        [jnp.zeros((1, 8), jnp.float32), mx_ref[:, 0:8]], axis=1)


def _prep(x, W, A):
    n, d_in = x.shape
    d = W.shape[1]
    nblk = n // ROW_BLK
    return pl.pallas_call(
        functools.partial(_prep_body, nblk),
        grid=(nblk,),
        in_specs=[
            pl.BlockSpec((ROW_BLK, d_in), lambda i: (i, 0)),
            pl.BlockSpec((d_in, d), lambda i: (0, 0)),
            pl.BlockSpec((d, 32), lambda i: (0, 0)),
        ],
        out_specs=[
            pl.BlockSpec((ROW_BLK, d + 16), lambda i: (i, 0)),
            pl.BlockSpec((ROW_BLK, 16), lambda i: (i, 0)),
            pl.BlockSpec((1, 16), lambda i: (0, 0)),
        ],
        out_shape=[
            jax.ShapeDtypeStruct((n, d + 16), jnp.float32),
            jax.ShapeDtypeStruct((n, 16), jnp.float32),
            jax.ShapeDtypeStruct((1, 16), jnp.float32),
        ],
        scratch_shapes=[pltpu.VMEM((1, 16), jnp.float32)],
    )(x, W, A)


def _att_matrix(att_src, att_dst, heads, ch):
    d = heads * ch
    A = jnp.zeros((d, 32), jnp.float32)
    hh = jnp.arange(heads)
    rows = (hh[:, None] * ch + jnp.arange(ch)[None, :]).reshape(-1)
    cols = jnp.repeat(hh, ch)
    asrc = att_src.reshape(-1)
    adst = att_dst.reshape(-1)
    A = A.at[rows, cols].set(asrc)
    A = A.at[rows, cols + 16].set(adst)
    A = A.at[rows, cols + 24].set(adst)
    return A


def _take16(v, idx):
    dn = lax.GatherDimensionNumbers(
        offset_dims=(), collapsed_slice_dims=(0,), start_index_map=(0,))
    return lax.gather(v, idx[:, None], dn, (1,),
                      mode=lax.GatherScatterMode.PROMISE_IN_BOUNDS)


def _make_edge_kernel(D, H):
    ROW = D + 16
    NV = D // 16
    STEPS = T_EDGE // K_EDGE
    mesh = plsc.VectorSubcoreMesh(core_axis_name="c", subcore_axis_name="s")

    @functools.partial(
        pl.kernel,
        out_type=jax.ShapeDtypeStruct((2, N_ACC, ROW), jnp.float32),
        mesh=mesh,
        scratch_types=[
            pltpu.VMEM((K_EDGE,), jnp.int32),
            pltpu.VMEM((K_EDGE,), jnp.int32),
            pltpu.VMEM((K_EDGE, ROW), jnp.float32),
            pltpu.VMEM((K_EDGE, 16), jnp.float32),
            pltpu.VMEM((1, 16), jnp.float32),
            pltpu.VMEM_SHARED((N_ACC, ROW), jnp.float32),
        ],
        compiler_params=pltpu.CompilerParams(use_tc_tiling_on_sc=False),
    )
    def edge_kernel(hs_hbm, adst_hbm, cvec_hbm, sidx_hbm, didx_hbm, zeros_hbm,
                    out_hbm, sidx_v, didx_v, rows_v, adst_v, cvec_v, acc_sh):
        cid = lax.axis_index("c")
        sid = lax.axis_index("s")
        wid = sid * 2 + cid
        rpc = N_ACC // 16
        pltpu.sync_copy(zeros_hbm.at[pl.ds(sid * rpc, rpc)],
                        acc_sh.at[pl.ds(sid * rpc, rpc)])
        pltpu.sync_copy(cvec_hbm, cvec_v)
        plsc.subcore_barrier()
        cv = cvec_v[0]
        iot = lax.iota(jnp.int32, 16)
        shift_idx = (iot & 7) + 8
        base0 = wid * T_EDGE

        @pl.loop(0, STEPS)
        def _(st):
            base = base0 + st * K_EDGE
            pltpu.sync_copy(sidx_hbm.at[pl.ds(base, K_EDGE)], sidx_v)
            pltpu.sync_copy(didx_hbm.at[pl.ds(base, K_EDGE)], didx_v)
            pltpu.sync_copy(hs_hbm.at[sidx_v], rows_v)
            pltpu.sync_copy(adst_hbm.at[didx_v], adst_v)

            @pl.loop(0, K_EDGE)
            def _(e):
                v1 = rows_v[e, pl.ds(D, 16)]
                v2 = adst_v[e] + cv
                t = v1 + v2
                al = jnp.where(t > 0, t, t * 0.2)
                mm = _take16(al, shift_idx)
                w16 = jnp.exp(al - mm)
                rows_v[e, pl.ds(D, 16)] = w16
                for j in range(NV):
                    hj = j if H > 1 else 0
                    ws = _take16(w16, jnp.full((16,), hj, jnp.int32))
                    rows_v[e, pl.ds(j * 16, 16)] = (
                        rows_v[e, pl.ds(j * 16, 16)] * ws)

            pltpu.sync_copy(rows_v, acc_sh.at[didx_v], add=True)

        plsc.subcore_barrier()
        pltpu.sync_copy(acc_sh.at[pl.ds(sid * rpc, rpc)],
                        out_hbm.at[cid, pl.ds(sid * rpc, rpc)])

    return edge_kernel


_edge_kernel_l1 = _make_edge_kernel(128, 8)
_edge_kernel_l2 = _make_edge_kernel(64, 1)


def _epi1_body(acc_ref, r_ref, b_ref, w2_ref, a2_ref,
               hs2_ref, adst2_ref, cvec2_ref, mx_ref):
    i = pl.program_id(0)
    comb = acc_ref[0] + acc_ref[1]
    num = comb[:, :128]
    den = jnp.dot(comb[:, 128:144], r_ref[...],
                  preferred_element_type=jnp.float32)
    pre = num / (den + 1e-16) + b_ref[...]
    x2 = jnp.where(pre > 0, pre, jnp.exp(pre) - 1.0)
    h2 = jnp.dot(x2, w2_ref[...], preferred_element_type=jnp.float32)
    ap2 = jnp.dot(h2, a2_ref[...], preferred_element_type=jnp.float32)
    hs2_ref[:, :64] = h2
    hs2_ref[:, 64:80] = ap2[:, 0:16]
    adst2_ref[...] = ap2[:, 16:32]
    blkmax = jnp.max(ap2[:, 0:16], axis=0, keepdims=True)

    @pl.when(i == 0)
    def _():
        mx_ref[...] = blkmax

    @pl.when(i > 0)
    def _():
        mx_ref[...] = jnp.maximum(mx_ref[...], blkmax)

    cvec2_ref[...] = jnp.concatenate(
        [jnp.zeros((1, 8), jnp.float32), mx_ref[:, 0:8]], axis=1)


def _epi1(acc1, R1, b1row, W2, A2):
    nblk = N_NODES // ROW_BLK
    return pl.pallas_call(
        _epi1_body,
        grid=(nblk,),
        in_specs=[
            pl.BlockSpec((2, ROW_BLK, 144), lambda i: (0, i, 0)),
            pl.BlockSpec((16, 128), lambda i: (0, 0)),
            pl.BlockSpec((1, 128), lambda i: (0, 0)),
            pl.BlockSpec((128, 64), lambda i: (0, 0)),
            pl.BlockSpec((64, 32), lambda i: (0, 0)),
        ],
        out_specs=[
            pl.BlockSpec((ROW_BLK, 80), lambda i: (i, 0)),
            pl.BlockSpec((ROW_BLK, 16), lambda i: (i, 0)),
            pl.BlockSpec((1, 16), lambda i: (0, 0)),
        ],
        out_shape=[
            jax.ShapeDtypeStruct((N_NODES, 80), jnp.float32),
            jax.ShapeDtypeStruct((N_NODES, 16), jnp.float32),
            jax.ShapeDtypeStruct((1, 16), jnp.float32),
        ],
        scratch_shapes=[pltpu.VMEM((1, 16), jnp.float32)],
    )(acc1, R1, b1row, W2, A2)


def _epi2_body(acc_ref, r_ref, b_ref, out_ref):
    comb = acc_ref[0] + acc_ref[1]
    num = comb[:, :64]
    den = jnp.dot(comb[:, 64:80], r_ref[...],
                  preferred_element_type=jnp.float32)
    out_ref[...] = num / (den + 1e-16) + b_ref[...]


def _epi2(acc2, R2, b2row):
    nblk = N_NODES // ROW_BLK
    return pl.pallas_call(
        _epi2_body,
        grid=(nblk,),
        in_specs=[
            pl.BlockSpec((2, ROW_BLK, 80), lambda i: (0, i, 0)),
            pl.BlockSpec((16, 64), lambda i: (0, 0)),
            pl.BlockSpec((1, 64), lambda i: (0, 0)),
        ],
        out_specs=pl.BlockSpec((ROW_BLK, 64), lambda i: (i, 0)),
        out_shape=jax.ShapeDtypeStruct((N_NODES, 64), jnp.float32),
    )(acc2, R2, b2row)


def _bcast_matrix(heads, ch, d):
    R = jnp.zeros((16, d), jnp.float32)
    hh = jnp.repeat(jnp.arange(heads), ch)
    R = R.at[hh, jnp.arange(d)].set(1.0)
    return R


def kernel(edge_index, node_emb, W1, att_src1, att_dst1, b1, W2, att_src2,
           att_dst2, b2):
    loops = jnp.arange(N_NODES, dtype=edge_index.dtype)
    ei = jnp.concatenate([edge_index, jnp.stack([loops, loops])], axis=1)
    ei = ei.astype(jnp.int32)
    pad = E_PAD - E_TOT
    sidx = jnp.concatenate([ei[0], jnp.zeros((pad,), jnp.int32)])
    didx = jnp.concatenate([ei[1], jnp.full((pad,), TRASH, jnp.int32)])

    A1 = _att_matrix(att_src1, att_dst1, 8, 16)
    A2 = _att_matrix(att_src2, att_dst2, 1, 64)
    R1 = _bcast_matrix(8, 16, 128)
    R2 = _bcast_matrix(1, 64, 64)
    zeros1 = jnp.zeros((N_ACC, 144), jnp.float32)
    zeros2 = jnp.zeros((N_ACC, 80), jnp.float32)

    hs1, adst1, cvec1 = _prep(node_emb, W1, A1)
    adst1p = jnp.concatenate(
        [adst1, jnp.zeros((N_ACC - N_NODES, 16), jnp.float32)])
    acc1 = _edge_kernel_l1(hs1, adst1p, cvec1, sidx, didx, zeros1)

    hs2, adst2, cvec2 = _epi1(acc1, R1, b1.reshape(1, 128), W2, A2)
    adst2p = jnp.concatenate(
        [adst2, jnp.zeros((N_ACC - N_NODES, 16), jnp.float32)])
    acc2 = _edge_kernel_l2(hs2, adst2p, cvec2, sidx, didx, zeros2)

    return _epi2(acc2, R2, b2.reshape(1, 64))


# speedup vs baseline: 54.5357x; 1.3079x over previous
import functools

import jax
import jax.numpy as jnp
from jax import lax
from jax.experimental import pallas as pl
from jax.experimental.pallas import tpu as pltpu
from jax.experimental.pallas import tpu_sc as plsc

N_NODES = 10000
N_EDGES = 320000
E_TOT = N_EDGES + N_NODES
NW = 32
K_EDGE = 112
STEPS = 94
T_EDGE = STEPS * K_EDGE
E_PAD = NW * T_EDGE
N_ACC = 10000
ROW_BLK = 1000


def _prep_body(nblk, x_ref, w_ref, a_ref, hs_ref, adst_ref, cvec_ref, mx_ref):
    i = pl.program_id(0)
    d = w_ref.shape[1]
    h = jnp.dot(x_ref[...], w_ref[...], preferred_element_type=jnp.float32)
    ap = jnp.dot(h, a_ref[...], preferred_element_type=jnp.float32)
    hs_ref[:, :d] = h
    hs_ref[:, d:d + 16] = ap[:, 0:16]
    adst_ref[...] = ap[:, 16:32]
    blkmax = jnp.max(ap[:, 0:16], axis=0, keepdims=True)

    @pl.when(i == 0)
    def _():
        mx_ref[...] = blkmax

    @pl.when(i > 0)
    def _():
        mx_ref[...] = jnp.maximum(mx_ref[...], blkmax)

    cvec_ref[...] = jnp.concatenate(
        [jnp.zeros((1, 8), jnp.float32), mx_ref[:, 0:8]], axis=1)


def _prep(x, W, A):
    n, d_in = x.shape
    d = W.shape[1]
    nblk = n // ROW_BLK
    return pl.pallas_call(
        functools.partial(_prep_body, nblk),
        grid=(nblk,),
        in_specs=[
            pl.BlockSpec((ROW_BLK, d_in), lambda i: (i, 0)),
            pl.BlockSpec((d_in, d), lambda i: (0, 0)),
            pl.BlockSpec((d, 32), lambda i: (0, 0)),
        ],
        out_specs=[
            pl.BlockSpec((ROW_BLK, d + 16), lambda i: (i, 0)),
            pl.BlockSpec((ROW_BLK, 16), lambda i: (i, 0)),
            pl.BlockSpec((1, 16), lambda i: (0, 0)),
        ],
        out_shape=[
            jax.ShapeDtypeStruct((n, d + 16), jnp.float32),
            jax.ShapeDtypeStruct((n, 16), jnp.float32),
            jax.ShapeDtypeStruct((1, 16), jnp.float32),
        ],
        scratch_shapes=[pltpu.VMEM((1, 16), jnp.float32)],
    )(x, W, A)


def _att_matrix(att_src, att_dst, heads, ch):
    d = heads * ch
    A = jnp.zeros((d, 32), jnp.float32)
    hh = jnp.arange(heads)
    rows = (hh[:, None] * ch + jnp.arange(ch)[None, :]).reshape(-1)
    cols = jnp.repeat(hh, ch)
    asrc = att_src.reshape(-1)
    adst = att_dst.reshape(-1)
    A = A.at[rows, cols].set(asrc)
    A = A.at[rows, cols + 16].set(adst)
    A = A.at[rows, cols + 24].set(adst)
    return A


def _take16(v, idx):
    dn = lax.GatherDimensionNumbers(
        offset_dims=(), collapsed_slice_dims=(0,), start_index_map=(0,))
    return lax.gather(v, idx[:, None], dn, (1,),
                      mode=lax.GatherScatterMode.PROMISE_IN_BOUNDS)


def _make_edge_kernel(D, H):
    ROW = D + 16
    NV = D // 16
    mesh = plsc.VectorSubcoreMesh(core_axis_name="c", subcore_axis_name="s")

    @functools.partial(
        pl.kernel,
        out_type=jax.ShapeDtypeStruct((2, N_ACC, ROW), jnp.float32),
        mesh=mesh,
        scratch_types=[
            pltpu.VMEM((K_EDGE,), jnp.int32),
            pltpu.VMEM((K_EDGE,), jnp.int32),
            pltpu.VMEM((K_EDGE,), jnp.int32),
            pltpu.VMEM((K_EDGE,), jnp.int32),
            pltpu.VMEM((K_EDGE, ROW), jnp.float32),
            pltpu.VMEM((K_EDGE, ROW), jnp.float32),
            pltpu.VMEM((K_EDGE, 16), jnp.float32),
            pltpu.VMEM((K_EDGE, 16), jnp.float32),
            pltpu.VMEM((1, 16), jnp.float32),
            pltpu.VMEM_SHARED((N_ACC, ROW), jnp.float32),
            pltpu.SemaphoreType.DMA,
            pltpu.SemaphoreType.DMA,
            pltpu.SemaphoreType.DMA,
            pltpu.SemaphoreType.DMA,
            pltpu.SemaphoreType.DMA,
            pltpu.SemaphoreType.DMA,
        ],
        compiler_params=pltpu.CompilerParams(use_tc_tiling_on_sc=False),
    )
    def edge_kernel(hs_hbm, adst_hbm, cvec_hbm, sidx_hbm, didx_hbm, zeros_hbm,
                    out_hbm, si0, si1, di0, di1, rows0, rows1, ad0, ad1,
                    cvec_v, acc_sh, sr0, sr1, sa0, sa1, ss0, ss1):
        sidx = (si0, si1)
        didx = (di0, di1)
        rows = (rows0, rows1)
        ads = (ad0, ad1)
        sem_r = (sr0, sr1)
        sem_a = (sa0, sa1)
        sem_s = (ss0, ss1)
        cid = lax.axis_index("c")
        sid = lax.axis_index("s")
        wid = sid * 2 + cid
        rpc = 624
        pltpu.sync_copy(zeros_hbm.at[pl.ds(0, rpc)],
                        acc_sh.at[pl.ds(sid * rpc, rpc)])

        @pl.when(sid == 15)
        def _():
            pltpu.sync_copy(zeros_hbm.at[pl.ds(0, 16)],
                            acc_sh.at[pl.ds(9984, 16)])
        pltpu.sync_copy(cvec_hbm, cvec_v)
        plsc.subcore_barrier()
        cv = cvec_v[0]
        iot = lax.iota(jnp.int32, 16)
        shift_idx = (iot & 7) + 8

        base0 = wid * T_EDGE

        def issue_gather(st, b):
            base = base0 + st * K_EDGE
            pltpu.sync_copy(sidx_hbm.at[pl.ds(base, K_EDGE)], sidx[b])
            pltpu.sync_copy(didx_hbm.at[pl.ds(base, K_EDGE)], didx[b])
            pltpu.async_copy(hs_hbm.at[sidx[b]], rows[b], sem_r[b])
            pltpu.async_copy(adst_hbm.at[didx[b]], ads[b], sem_a[b])

        def wait_gather(st, b):
            pltpu.make_async_copy(hs_hbm.at[sidx[b]], rows[b],
                                  sem_r[b]).wait()
            pltpu.make_async_copy(adst_hbm.at[didx[b]], ads[b],
                                  sem_a[b]).wait()

        issue_gather(0, 0)

        @pl.loop(0, STEPS // 2)
        def _(it):
            for b in range(2):
                st = it * 2 + b
                rv = rows[b]

                @pl.when(st + 1 < STEPS)
                def _():
                    issue_gather(st + 1, 1 - b)

                wait_gather(st, b)

                @pl.loop(0, K_EDGE)
                def _(e):
                    v1 = rv[e, pl.ds(D, 16)]
                    v2 = ads[b][e] + cv
                    t = v1 + v2
                    al = jnp.where(t > 0, t, t * 0.2)
                    mm = _take16(al, shift_idx)
                    w16 = jnp.exp(al - mm)
                    rv[e, pl.ds(D, 16)] = w16
                    for j in range(NV):
                        hj = j if H > 1 else 0
                        ws = _take16(w16, jnp.full((16,), hj, jnp.int32))
                        rv[e, pl.ds(j * 16, 16)] = (
                            rv[e, pl.ds(j * 16, 16)] * ws)

                pltpu.sync_copy(rv, acc_sh.at[didx[b]], add=True)

        plsc.subcore_barrier()
        pltpu.sync_copy(acc_sh.at[pl.ds(sid * rpc, rpc)],
                        out_hbm.at[cid, pl.ds(sid * rpc, rpc)])

        @pl.when(sid == 15)
        def _():
            pltpu.sync_copy(acc_sh.at[pl.ds(9984, 16)],
                            out_hbm.at[cid, pl.ds(9984, 16)])

    return edge_kernel


_edge_kernel_l1 = _make_edge_kernel(128, 8)
_edge_kernel_l2 = _make_edge_kernel(64, 1)


def _epi1_body(acc_ref, r_ref, b_ref, w2_ref, a2_ref,
               hs2_ref, adst2_ref, cvec2_ref, mx_ref):
    i = pl.program_id(0)
    comb = acc_ref[0] + acc_ref[1]
    num = comb[:, :128]
    den = jnp.dot(comb[:, 128:144], r_ref[...],
                  preferred_element_type=jnp.float32)
    pre = num / (den + 1e-16) + b_ref[...]
    x2 = jnp.where(pre > 0, pre, jnp.exp(pre) - 1.0)
    h2 = jnp.dot(x2, w2_ref[...], preferred_element_type=jnp.float32)
    ap2 = jnp.dot(h2, a2_ref[...], preferred_element_type=jnp.float32)
    hs2_ref[:, :64] = h2
    hs2_ref[:, 64:80] = ap2[:, 0:16]
    adst2_ref[...] = ap2[:, 16:32]
    blkmax = jnp.max(ap2[:, 0:16], axis=0, keepdims=True)

    @pl.when(i == 0)
    def _():
        mx_ref[...] = blkmax

    @pl.when(i > 0)
    def _():
        mx_ref[...] = jnp.maximum(mx_ref[...], blkmax)

    cvec2_ref[...] = jnp.concatenate(
        [jnp.zeros((1, 8), jnp.float32), mx_ref[:, 0:8]], axis=1)


def _epi1(acc1, R1, b1row, W2, A2):
    nblk = N_NODES // ROW_BLK
    return pl.pallas_call(
        _epi1_body,
        grid=(nblk,),
        in_specs=[
            pl.BlockSpec((2, ROW_BLK, 144), lambda i: (0, i, 0)),
            pl.BlockSpec((16, 128), lambda i: (0, 0)),
            pl.BlockSpec((1, 128), lambda i: (0, 0)),
            pl.BlockSpec((128, 64), lambda i: (0, 0)),
            pl.BlockSpec((64, 32), lambda i: (0, 0)),
        ],
        out_specs=[
            pl.BlockSpec((ROW_BLK, 80), lambda i: (i, 0)),
            pl.BlockSpec((ROW_BLK, 16), lambda i: (i, 0)),
            pl.BlockSpec((1, 16), lambda i: (0, 0)),
        ],
        out_shape=[
            jax.ShapeDtypeStruct((N_NODES, 80), jnp.float32),
            jax.ShapeDtypeStruct((N_NODES, 16), jnp.float32),
            jax.ShapeDtypeStruct((1, 16), jnp.float32),
        ],
        scratch_shapes=[pltpu.VMEM((1, 16), jnp.float32)],
    )(acc1, R1, b1row, W2, A2)


def _epi2_body(acc_ref, r_ref, b_ref, out_ref):
    comb = acc_ref[0] + acc_ref[1]
    num = comb[:, :64]
    den = jnp.dot(comb[:, 64:80], r_ref[...],
                  preferred_element_type=jnp.float32)
    out_ref[...] = num / (den + 1e-16) + b_ref[...]


def _epi2(acc2, R2, b2row):
    nblk = N_NODES // ROW_BLK
    return pl.pallas_call(
        _epi2_body,
        grid=(nblk,),
        in_specs=[
            pl.BlockSpec((2, ROW_BLK, 80), lambda i: (0, i, 0)),
            pl.BlockSpec((16, 64), lambda i: (0, 0)),
            pl.BlockSpec((1, 64), lambda i: (0, 0)),
        ],
        out_specs=pl.BlockSpec((ROW_BLK, 64), lambda i: (i, 0)),
        out_shape=jax.ShapeDtypeStruct((N_NODES, 64), jnp.float32),
    )(acc2, R2, b2row)


def _bcast_matrix(heads, ch, d):
    R = jnp.zeros((16, d), jnp.float32)
    hh = jnp.repeat(jnp.arange(heads), ch)
    R = R.at[hh, jnp.arange(d)].set(1.0)
    return R


def _sentinel_row(d):
    r = jnp.zeros((1, d + 16), jnp.float32)
    return r.at[0, d:d + 8].set(-1e30)


def kernel(edge_index, node_emb, W1, att_src1, att_dst1, b1, W2, att_src2,
           att_dst2, b2):
    loops = jnp.arange(N_NODES, dtype=edge_index.dtype)
    ei = jnp.concatenate([edge_index, jnp.stack([loops, loops])], axis=1)
    ei = ei.astype(jnp.int32)
    pad = E_PAD - E_TOT
    sidx = jnp.concatenate([ei[0], jnp.full((pad,), N_NODES, jnp.int32)])
    didx = jnp.concatenate([ei[1], jnp.zeros((pad,), jnp.int32)])

    A1 = _att_matrix(att_src1, att_dst1, 8, 16)
    A2 = _att_matrix(att_src2, att_dst2, 1, 64)
    R1 = _bcast_matrix(8, 16, 128)
    R2 = _bcast_matrix(1, 64, 64)
    zeros1 = jnp.zeros((624, 144), jnp.float32)
    zeros2 = jnp.zeros((624, 80), jnp.float32)

    hs1, adst1, cvec1 = _prep(node_emb, W1, A1)
    hs1 = jnp.concatenate([hs1, _sentinel_row(128)])
    acc1 = _edge_kernel_l1(hs1, adst1, cvec1, sidx, didx, zeros1)

    hs2, adst2, cvec2 = _epi1(acc1, R1, b1.reshape(1, 128), W2, A2)
    hs2 = jnp.concatenate([hs2, _sentinel_row(64)])
    acc2 = _edge_kernel_l2(hs2, adst2, cvec2, sidx, didx, zeros2)

    return _epi2(acc2, R2, b2.reshape(1, 64))
